# Initial kernel scaffold; baseline (speedup 1.0000x reference)
import jax
import jax.numpy as jnp
from jax.experimental import pallas as pl


def kernel(x, ln1_w, ln2_w, wq, wk, wv, wo, q_norm_w, k_norm_w, router_w, gate_w, up_w, down_w):
    raise NotImplementedError("write your pallas kernel here")


# speedup vs baseline: 1.3044x; 1.3044x over previous
import functools

import jax
import jax.numpy as jnp
from jax.experimental import pallas as pl
from jax.experimental.pallas import tpu as pltpu

_EPS = 1e-06
_THETA = 10000.0
_NEG = -1e9


def _rms(x, w):
    var = jnp.mean(x * x, axis=-1, keepdims=True)
    return w * (x * jax.lax.rsqrt(var + _EPS))


def _dot_t(a, b):
    return jax.lax.dot_general(a, b, (((1,), (1,)), ((), ())),
                               preferred_element_type=jnp.float32)


def _qkv_kernel(x_ref, ln1_ref, wq_ref, wk_ref, wv_ref, qn_ref, kn_ref,
                q_ref, k_ref, v_ref, *, bs, hd, nh, nkv):
    sb = pl.program_id(1)
    xb = x_ref[0]
    h = _rms(xb, ln1_ref[...])
    q = _dot_t(h, wq_ref[...])
    k = _dot_t(h, wk_ref[...])
    v = _dot_t(h, wv_ref[...])

    half = hd // 2
    pos = sb * bs + jax.lax.broadcasted_iota(jnp.int32, (bs, 1), 0).astype(jnp.float32)
    j2 = 2.0 * jax.lax.broadcasted_iota(jnp.int32, (1, half), 1).astype(jnp.float32)
    inv = _THETA ** (-j2 / hd)
    ang = pos * inv
    c = jnp.cos(ang)
    s = jnp.sin(ang)

    def rope(t):
        x1 = t[:, :half]
        x2 = t[:, half:]
        return jnp.concatenate([x1 * c - x2 * s, x2 * c + x1 * s], axis=-1)

    for i in range(nh):
        qh = _rms(q[:, i * hd:(i + 1) * hd], qn_ref[...])
        q_ref[0, i] = rope(qh)
    for i in range(nkv):
        kh = _rms(k[:, i * hd:(i + 1) * hd], kn_ref[...])
        k_ref[0, i] = rope(kh)
        v_ref[0, i] = v[:, i * hd:(i + 1) * hd]


def _attn_kernel(q_ref, k_ref, v_ref, o_ref, *, bq, seq, hd):
    qi = pl.program_id(2)
    q = q_ref[0, 0]
    k = k_ref[0, 0]
    v = v_ref[0, 0]
    s = _dot_t(q, k) * (hd ** -0.5)
    row = qi * bq + jax.lax.broadcasted_iota(jnp.int32, (bq, seq), 0)
    col = jax.lax.broadcasted_iota(jnp.int32, (bq, seq), 1)
    s = jnp.where(col <= row, s, _NEG)
    m = jnp.max(s, axis=-1, keepdims=True)
    p = jnp.exp(s - m)
    p = p / jnp.sum(p, axis=-1, keepdims=True)
    o_ref[0, 0] = jax.lax.dot_general(p, v, (((1,), (0,)), ((), ())),
                                      preferred_element_type=jnp.float32)


def _post_kernel(o_ref, x_ref, ln2_ref, wo_ref, rw_ref,
                 x2_ref, h2_ref, lg_ref, *, nh):
    o = jnp.concatenate([o_ref[0, i] for i in range(nh)], axis=-1)
    x2 = x_ref[0] + _dot_t(o, wo_ref[...])
    h2 = _rms(x2, ln2_ref[...])
    x2_ref[0] = x2
    h2_ref[...] = h2
    lg_ref[...] = _dot_t(h2, rw_ref[...])


def _route_kernel(lg_ref, wd_ref, *, ne):
    z = lg_ref[...]
    z = z - jnp.max(z, axis=-1, keepdims=True)
    p = jnp.exp(z)
    p = p / jnp.sum(p, axis=-1, keepdims=True)
    eio = jax.lax.broadcasted_iota(jnp.int32, p.shape, 1)
    m1 = jnp.max(p, axis=-1, keepdims=True)
    i1 = jnp.min(jnp.where(p == m1, eio, ne), axis=-1, keepdims=True)
    oh1 = eio == i1
    p2 = jnp.where(oh1, -1.0, p)
    m2 = jnp.max(p2, axis=-1, keepdims=True)
    i2 = jnp.min(jnp.where(p2 == m2, eio, ne), axis=-1, keepdims=True)
    oh2 = eio == i2
    denom = m1 + m2
    wd_ref[...] = (jnp.where(oh1, m1, 0.0) + jnp.where(oh2, m2, 0.0)) / denom


def _moe_kernel(h2_ref, wd_ref, x2f_ref, gw_ref, uw_ref, dw_ref, out_ref,
                *, ne):
    e = pl.program_id(1)
    t = h2_ref[...]
    g = _dot_t(t, gw_ref[0])
    u = _dot_t(t, uw_ref[0])
    he = g * jax.lax.logistic(g) * u
    ye = _dot_t(he, dw_ref[0])
    eio = jax.lax.broadcasted_iota(jnp.int32, wd_ref.shape, 1)
    w = jnp.sum(jnp.where(eio == e, wd_ref[...], 0.0), axis=-1, keepdims=True)

    @pl.when(e == 0)
    def _():
        out_ref[...] = x2f_ref[...] + w * ye

    @pl.when(e > 0)
    def _():
        out_ref[...] += w * ye


def kernel(x, ln1_w, ln2_w, wq, wk, wv, wo, q_norm_w, k_norm_w,
           router_w, gate_w, up_w, down_w):
    B, S, D = x.shape
    HD = q_norm_w.shape[0]
    NH = wq.shape[0] // HD
    NKV = wk.shape[0] // HD
    E, FF, _ = gate_w.shape
    T = B * S
    BS = min(256, S)
    NSB = S // BS

    ln1 = ln1_w.reshape(1, D)
    ln2 = ln2_w.reshape(1, D)
    qn = q_norm_w.reshape(1, HD)
    kn = k_norm_w.reshape(1, HD)

    f32 = jnp.float32

    q, k, v = pl.pallas_call(
        functools.partial(_qkv_kernel, bs=BS, hd=HD, nh=NH, nkv=NKV),
        grid=(B, NSB),
        in_specs=[
            pl.BlockSpec((1, BS, D), lambda b, s: (b, s, 0)),
            pl.BlockSpec((1, D), lambda b, s: (0, 0)),
            pl.BlockSpec(wq.shape, lambda b, s: (0, 0)),
            pl.BlockSpec(wk.shape, lambda b, s: (0, 0)),
            pl.BlockSpec(wv.shape, lambda b, s: (0, 0)),
            pl.BlockSpec((1, HD), lambda b, s: (0, 0)),
            pl.BlockSpec((1, HD), lambda b, s: (0, 0)),
        ],
        out_specs=[
            pl.BlockSpec((1, NH, BS, HD), lambda b, s: (b, 0, s, 0)),
            pl.BlockSpec((1, NKV, BS, HD), lambda b, s: (b, 0, s, 0)),
            pl.BlockSpec((1, NKV, BS, HD), lambda b, s: (b, 0, s, 0)),
        ],
        out_shape=[
            jax.ShapeDtypeStruct((B, NH, S, HD), f32),
            jax.ShapeDtypeStruct((B, NKV, S, HD), f32),
            jax.ShapeDtypeStruct((B, NKV, S, HD), f32),
        ],
    )(x, ln1, wq, wk, wv, qn, kn)

    BQ = min(256, S)
    o = pl.pallas_call(
        functools.partial(_attn_kernel, bq=BQ, seq=S, hd=HD),
        grid=(B, NH, S // BQ),
        in_specs=[
            pl.BlockSpec((1, 1, BQ, HD), lambda b, h, i: (b, h, i, 0)),
            pl.BlockSpec((1, 1, S, HD), lambda b, h, i: (b, h, 0, 0)),
            pl.BlockSpec((1, 1, S, HD), lambda b, h, i: (b, h, 0, 0)),
        ],
        out_specs=pl.BlockSpec((1, 1, BQ, HD), lambda b, h, i: (b, h, i, 0)),
        out_shape=jax.ShapeDtypeStruct((B, NH, S, HD), f32),
    )(q, k, v)

    x2, h2, logits = pl.pallas_call(
        functools.partial(_post_kernel, nh=NH),
        grid=(B, NSB),
        in_specs=[
            pl.BlockSpec((1, NH, BS, HD), lambda b, s: (b, 0, s, 0)),
            pl.BlockSpec((1, BS, D), lambda b, s: (b, s, 0)),
            pl.BlockSpec((1, D), lambda b, s: (0, 0)),
            pl.BlockSpec(wo.shape, lambda b, s: (0, 0)),
            pl.BlockSpec(router_w.shape, lambda b, s: (0, 0)),
        ],
        out_specs=[
            pl.BlockSpec((1, BS, D), lambda b, s: (b, s, 0)),
            pl.BlockSpec((BS, D), lambda b, s: (b * NSB + s, 0)),
            pl.BlockSpec((BS, E), lambda b, s: (b * NSB + s, 0)),
        ],
        out_shape=[
            jax.ShapeDtypeStruct((B, S, D), f32),
            jax.ShapeDtypeStruct((T, D), f32),
            jax.ShapeDtypeStruct((T, E), f32),
        ],
    )(o, x, ln2, wo, router_w)

    wdense = pl.pallas_call(
        functools.partial(_route_kernel, ne=E),
        grid=(T // BS,),
        in_specs=[pl.BlockSpec((BS, E), lambda i: (i, 0))],
        out_specs=pl.BlockSpec((BS, E), lambda i: (i, 0)),
        out_shape=jax.ShapeDtypeStruct((T, E), f32),
    )(logits)

    BT = min(1024, T)
    x2f = x2.reshape(T, D)
    moe = pl.pallas_call(
        functools.partial(_moe_kernel, ne=E),
        grid=(T // BT, E),
        in_specs=[
            pl.BlockSpec((BT, D), lambda t, e: (t, 0)),
            pl.BlockSpec((BT, E), lambda t, e: (t, 0)),
            pl.BlockSpec((BT, D), lambda t, e: (t, 0)),
            pl.BlockSpec((1, FF, D), lambda t, e: (e, 0, 0)),
            pl.BlockSpec((1, FF, D), lambda t, e: (e, 0, 0)),
            pl.BlockSpec((1, D, FF), lambda t, e: (e, 0, 0)),
        ],
        out_specs=pl.BlockSpec((BT, D), lambda t, e: (t, 0)),
        out_shape=jax.ShapeDtypeStruct((T, D), f32),
    )(h2, wdense, x2f, gate_w, up_w, down_w)

    return moe.reshape(B, S, D)
